# baseline (speedup 1.0000x reference)
import functools

import jax
import jax.numpy as jnp
from jax.experimental import pallas as pl
from jax.experimental.pallas import tpu as pltpu

_BETA = 0.25
_MB = 256


def _vq_body(z_ref, zbf_ref, zsq_ref, e_ref, esq_ref,
             loss_ref, zq_ref, idx_ref, perp_ref,
             counts_ref, lsum_ref, *, n_codes, n_total):
    i = pl.program_id(0)
    nsteps = pl.num_programs(0)

    @pl.when(i == 0)
    def _init():
        counts_ref[...] = jnp.zeros_like(counts_ref)
        lsum_ref[0] = 0.0

    zb = z_ref[...]
    zbf = zbf_ref[...]
    e = e_ref[...]
    zsq = zsq_ref[...]
    esq = esq_ref[...]

    mm = jax.lax.dot_general(zbf, e, (((1,), (1,)), ((), ())),
                             preferred_element_type=jnp.float32)
    d = (zsq + esq) - 2.0 * mm

    half = n_codes // 2
    jiota = jax.lax.broadcasted_iota(jnp.int32, (zb.shape[0], half), 1)

    d0 = d[:, :half]
    m0 = jnp.min(d0, axis=1, keepdims=True)
    i0 = jnp.min(jnp.where(d0 == m0, jiota, n_codes), axis=1)

    d1 = d[:, half:]
    m1 = jnp.min(d1, axis=1, keepdims=True)
    i1 = jnp.min(jnp.where(d1 == m1, jiota + half, n_codes), axis=1)

    m0_bf = m0.astype(jnp.bfloat16).astype(jnp.float32)
    idx = jnp.where((m1 < m0_bf)[:, 0], i1, i0).astype(jnp.int32)
    idx_ref[...] = idx

    onehot = (jax.lax.broadcasted_iota(jnp.int32, (zb.shape[0], n_codes), 1)
              == idx[:, None]).astype(jnp.float32)
    zq = jax.lax.dot_general(onehot, e, (((1,), (0,)), ((), ())),
                             preferred_element_type=jnp.float32)

    diff = zq - zb
    zq_ref[...] = zb + diff
    lsum_ref[0] += jnp.sum(diff * diff)
    counts_ref[...] += jnp.sum(onehot, axis=0)

    @pl.when(i == nsteps - 1)
    def _finalize():
        mean = lsum_ref[0] / n_total
        loss_ref[...] = jnp.full((1, 1), mean + _BETA * mean, jnp.float32)
        e_mean = counts_ref[...] * (1.0 / (n_total / e.shape[1]))
        ent = -jnp.sum(e_mean * jnp.log(e_mean + 1e-10))
        perp_ref[...] = jnp.full((1, 1), jnp.exp(ent), jnp.float32)


def kernel(z, embedding_weight):
    e_dim = z.shape[-1]
    zf = z.reshape(-1, e_dim)
    m = zf.shape[0]
    n = embedding_weight.shape[0]
    n_total = m * e_dim

    zbf = zf.astype(jnp.bfloat16)
    zsq = jnp.sum(zf ** 2, axis=1).reshape(m, 1)
    esq = jnp.sum(embedding_weight ** 2, axis=1).reshape(1, n)

    body = functools.partial(_vq_body, n_codes=n, n_total=n_total)
    loss2d, zq_st, idx, perp2d = pl.pallas_call(
        body,
        grid=(m // _MB,),
        in_specs=[
            pl.BlockSpec((_MB, e_dim), lambda i: (i, 0)),
            pl.BlockSpec((_MB, e_dim), lambda i: (i, 0)),
            pl.BlockSpec((_MB, 1), lambda i: (i, 0)),
            pl.BlockSpec((n, e_dim), lambda i: (0, 0)),
            pl.BlockSpec((1, n), lambda i: (0, 0)),
        ],
        out_specs=[
            pl.BlockSpec((1, 1), lambda i: (0, 0)),
            pl.BlockSpec((_MB, e_dim), lambda i: (i, 0)),
            pl.BlockSpec((_MB,), lambda i: (i,)),
            pl.BlockSpec((1, 1), lambda i: (0, 0)),
        ],
        out_shape=[
            jax.ShapeDtypeStruct((1, 1), jnp.float32),
            jax.ShapeDtypeStruct((m, e_dim), jnp.float32),
            jax.ShapeDtypeStruct((m,), jnp.int32),
            jax.ShapeDtypeStruct((1, 1), jnp.float32),
        ],
        scratch_shapes=[
            pltpu.VMEM((n,), jnp.float32),
            pltpu.SMEM((1,), jnp.float32),
        ],
    )(zf, zbf, zsq, embedding_weight, esq)

    loss = loss2d.reshape(())
    perplexity = perp2d.reshape(())
    return (loss, zq_st.reshape(z.shape), idx, perplexity)


# speedup vs baseline: 1.0568x; 1.0568x over previous
import functools

import jax
import jax.numpy as jnp
from jax.experimental import pallas as pl
from jax.experimental.pallas import tpu as pltpu

_BETA = 0.25
_MB = 256


def _vq_body(z_ref, zbf_ref, zsq_ref, e_ref, esq_ref,
             loss_ref, zq_ref, idx_ref, perp_ref,
             counts_ref, lsum_ref, *, n_codes, n_total):
    i = pl.program_id(0)
    nsteps = pl.num_programs(0)

    @pl.when(i == 0)
    def _init():
        counts_ref[...] = jnp.zeros_like(counts_ref)
        lsum_ref[0] = 0.0

    zb = z_ref[...]
    zbf = zbf_ref[...]
    e = e_ref[...]
    zsq = zsq_ref[...]
    esq = esq_ref[...]

    mm = jax.lax.dot_general(zbf, e, (((1,), (1,)), ((), ())),
                             preferred_element_type=jnp.float32)
    d = (zsq + esq) + mm

    half = n_codes // 2
    jiota = jax.lax.broadcasted_iota(jnp.int32, (zb.shape[0], half), 1)

    d0 = d[:, :half]
    m0 = jnp.min(d0, axis=1, keepdims=True)
    i0 = jnp.min(jnp.where(d0 == m0, jiota, n_codes), axis=1)

    d1 = d[:, half:]
    m1 = jnp.min(d1, axis=1, keepdims=True)
    i1 = jnp.min(jnp.where(d1 == m1, jiota + half, n_codes), axis=1)

    m0_bf = m0.astype(jnp.bfloat16).astype(jnp.float32)
    idx = jnp.where((m1 < m0_bf)[:, 0], i1, i0).astype(jnp.int32)
    idx_ref[...] = idx

    onehot = (jax.lax.broadcasted_iota(jnp.int32, (zb.shape[0], n_codes), 1)
              == idx[:, None]).astype(jnp.float32)
    zq = jax.lax.dot_general(onehot, e, (((1,), (0,)), ((), ())),
                             preferred_element_type=jnp.float32)

    diff = zq - zb
    zq_ref[...] = zb + diff
    lsum_ref[0] += jnp.sum(diff * diff)
    ones_row = jnp.ones((1, zb.shape[0]), jnp.float32)
    counts_ref[...] += jax.lax.dot_general(
        ones_row, onehot, (((1,), (0,)), ((), ())),
        preferred_element_type=jnp.float32)

    @pl.when(i == nsteps - 1)
    def _finalize():
        mean = lsum_ref[0] / n_total
        loss_ref[...] = jnp.full((1, 1), mean + _BETA * mean, jnp.float32)
        e_mean = counts_ref[...] * (1.0 / (n_total / e.shape[1]))
        ent = -jnp.sum(e_mean * jnp.log(e_mean + 1e-10))
        perp_ref[...] = jnp.full((1, 1), jnp.exp(ent), jnp.float32)


def kernel(z, embedding_weight):
    e_dim = z.shape[-1]
    zf = z.reshape(-1, e_dim)
    m = zf.shape[0]
    n = embedding_weight.shape[0]
    n_total = m * e_dim

    zbf = (-2.0 * zf).astype(jnp.bfloat16)
    zsq = jnp.sum(zf ** 2, axis=1).reshape(m, 1)
    esq = jnp.sum(embedding_weight ** 2, axis=1).reshape(1, n)

    body = functools.partial(_vq_body, n_codes=n, n_total=n_total)
    loss2d, zq_st, idx, perp2d = pl.pallas_call(
        body,
        grid=(m // _MB,),
        in_specs=[
            pl.BlockSpec((_MB, e_dim), lambda i: (i, 0)),
            pl.BlockSpec((_MB, e_dim), lambda i: (i, 0)),
            pl.BlockSpec((_MB, 1), lambda i: (i, 0)),
            pl.BlockSpec((n, e_dim), lambda i: (0, 0)),
            pl.BlockSpec((1, n), lambda i: (0, 0)),
        ],
        out_specs=[
            pl.BlockSpec((1, 1), lambda i: (0, 0)),
            pl.BlockSpec((_MB, e_dim), lambda i: (i, 0)),
            pl.BlockSpec((_MB,), lambda i: (i,)),
            pl.BlockSpec((1, 1), lambda i: (0, 0)),
        ],
        out_shape=[
            jax.ShapeDtypeStruct((1, 1), jnp.float32),
            jax.ShapeDtypeStruct((m, e_dim), jnp.float32),
            jax.ShapeDtypeStruct((m,), jnp.int32),
            jax.ShapeDtypeStruct((1, 1), jnp.float32),
        ],
        scratch_shapes=[
            pltpu.VMEM((1, n), jnp.float32),
            pltpu.SMEM((1,), jnp.float32),
        ],
    )(zf, zbf, zsq, embedding_weight, esq)

    loss = loss2d.reshape(())
    perplexity = perp2d.reshape(())
    return (loss, zq_st.reshape(z.shape), idx, perplexity)


# speedup vs baseline: 1.0817x; 1.0235x over previous
import functools

import jax
import jax.numpy as jnp
from jax.experimental import pallas as pl
from jax.experimental.pallas import tpu as pltpu

_BETA = 0.25
_MB = 512


def _vq_body(z_ref, zbf_ref, zsq_ref, e_ref, esq_ref,
             loss_ref, zq_ref, idx_ref, perp_ref,
             counts_ref, lsum_ref, *, n_codes, n_total):
    i = pl.program_id(0)
    nsteps = pl.num_programs(0)

    @pl.when(i == 0)
    def _init():
        counts_ref[...] = jnp.zeros_like(counts_ref)
        lsum_ref[0] = 0.0

    zb = z_ref[...]
    zbf = zbf_ref[...]
    e = e_ref[...]
    zsq = zsq_ref[...]
    esq = esq_ref[...]

    mm = jax.lax.dot_general(zbf, e, (((1,), (1,)), ((), ())),
                             preferred_element_type=jnp.float32)
    d = (zsq + esq) + mm

    half = n_codes // 2
    d0 = d[:, :half]
    d1 = d[:, half:]
    m0 = jnp.min(d0, axis=1, keepdims=True)
    m1 = jnp.min(d1, axis=1, keepdims=True)
    m0_bf = m0.astype(jnp.bfloat16).astype(jnp.float32)
    take = m1 < m0_bf

    d_w = jnp.where(take, d1, d0)
    m_w = jnp.where(take, m1, m0)
    jiota = jax.lax.broadcasted_iota(jnp.int32, (zb.shape[0], half), 1)
    i_rel = jnp.min(jnp.where(d_w == m_w, jiota, n_codes), axis=1)
    idx = (i_rel + jnp.where(take[:, 0], half, 0)).astype(jnp.int32)
    idx_ref[...] = idx

    onehot = (jax.lax.broadcasted_iota(jnp.int32, (zb.shape[0], n_codes), 1)
              == idx[:, None]).astype(jnp.float32)
    zq = jax.lax.dot_general(onehot, e, (((1,), (0,)), ((), ())),
                             preferred_element_type=jnp.float32)

    diff = zq - zb
    zq_ref[...] = zb + diff
    lsum_ref[0] += jnp.sum(diff * diff)
    ones_row = jnp.ones((1, zb.shape[0]), jnp.float32)
    counts_ref[...] += jax.lax.dot_general(
        ones_row, onehot, (((1,), (0,)), ((), ())),
        preferred_element_type=jnp.float32)

    @pl.when(i == nsteps - 1)
    def _finalize():
        mean = lsum_ref[0] / n_total
        loss_ref[...] = jnp.full((1, 1), mean + _BETA * mean, jnp.float32)
        e_mean = counts_ref[...] * (1.0 / (n_total / e.shape[1]))
        ent = -jnp.sum(e_mean * jnp.log(e_mean + 1e-10))
        perp_ref[...] = jnp.full((1, 1), jnp.exp(ent), jnp.float32)


def kernel(z, embedding_weight):
    e_dim = z.shape[-1]
    zf = z.reshape(-1, e_dim)
    m = zf.shape[0]
    n = embedding_weight.shape[0]
    n_total = m * e_dim

    zbf = (-2.0 * zf).astype(jnp.bfloat16)
    zsq = jnp.sum(zf ** 2, axis=1).reshape(m, 1)
    esq = jnp.sum(embedding_weight ** 2, axis=1).reshape(1, n)

    body = functools.partial(_vq_body, n_codes=n, n_total=n_total)
    loss2d, zq_st, idx, perp2d = pl.pallas_call(
        body,
        grid=(m // _MB,),
        in_specs=[
            pl.BlockSpec((_MB, e_dim), lambda i: (i, 0)),
            pl.BlockSpec((_MB, e_dim), lambda i: (i, 0)),
            pl.BlockSpec((_MB, 1), lambda i: (i, 0)),
            pl.BlockSpec((n, e_dim), lambda i: (0, 0)),
            pl.BlockSpec((1, n), lambda i: (0, 0)),
        ],
        out_specs=[
            pl.BlockSpec((1, 1), lambda i: (0, 0)),
            pl.BlockSpec((_MB, e_dim), lambda i: (i, 0)),
            pl.BlockSpec((_MB,), lambda i: (i,)),
            pl.BlockSpec((1, 1), lambda i: (0, 0)),
        ],
        out_shape=[
            jax.ShapeDtypeStruct((1, 1), jnp.float32),
            jax.ShapeDtypeStruct((m, e_dim), jnp.float32),
            jax.ShapeDtypeStruct((m,), jnp.int32),
            jax.ShapeDtypeStruct((1, 1), jnp.float32),
        ],
        scratch_shapes=[
            pltpu.VMEM((1, n), jnp.float32),
            pltpu.SMEM((1,), jnp.float32),
        ],
    )(zf, zbf, zsq, embedding_weight, esq)

    loss = loss2d.reshape(())
    perplexity = perp2d.reshape(())
    return (loss, zq_st.reshape(z.shape), idx, perplexity)


# speedup vs baseline: 1.2770x; 1.1806x over previous
import functools

import jax
import jax.numpy as jnp
from jax.experimental import pallas as pl
from jax.experimental.pallas import tpu as pltpu

_BETA = 0.25
_MB = 512


def _vq_body(z_ref, zbf_ref, zsq_ref, e_ref, esq_ref,
             loss_ref, zq_ref, idx_ref, perp_ref,
             counts_ref, lsum_ref, *, n_codes, n_total):
    i = pl.program_id(0)
    nsteps = pl.num_programs(0)

    @pl.when(i == 0)
    def _init():
        counts_ref[...] = jnp.zeros_like(counts_ref)
        lsum_ref[0] = 0.0

    zb = z_ref[...]
    zbf = zbf_ref[...]
    e = e_ref[...]
    zsq = zsq_ref[...]
    esq = esq_ref[...]

    mm = jax.lax.dot_general(zbf, e, (((1,), (1,)), ((), ())),
                             preferred_element_type=jnp.float32)
    d = (zsq + esq) + mm

    half = n_codes // 2
    d0 = d[:, :half]
    d1 = d[:, half:]
    m0 = jnp.min(d0, axis=1, keepdims=True)
    m1 = jnp.min(d1, axis=1, keepdims=True)
    m0_bf = m0.astype(jnp.bfloat16).astype(jnp.float32)
    take = m1 < m0_bf

    d_w = jnp.where(take, d1, d0)
    m_w = jnp.where(take, m1, m0)
    jiota = jax.lax.broadcasted_iota(jnp.int32, (zb.shape[0], half), 1)
    i_rel = jnp.min(jnp.where(d_w == m_w, jiota, n_codes), axis=1)
    idx = (i_rel + jnp.where(take[:, 0], half, 0)).astype(jnp.int32)
    idx_ref[...] = idx

    onehot = (jiota == i_rel[:, None]).astype(jnp.float32)
    zq0 = jax.lax.dot_general(onehot, e[:half], (((1,), (0,)), ((), ())),
                              preferred_element_type=jnp.float32)
    zq1 = jax.lax.dot_general(onehot, e[half:], (((1,), (0,)), ((), ())),
                              preferred_element_type=jnp.float32)
    zq = jnp.where(take, zq1, zq0)

    diff = zq - zb
    zq_ref[...] = zb + diff
    lsum_ref[0] += jnp.sum(diff * diff)
    take_row = take.astype(jnp.float32).reshape(1, zb.shape[0])
    keep_row = 1.0 - take_row
    counts_ref[0:1, :half] += jax.lax.dot_general(
        keep_row, onehot, (((1,), (0,)), ((), ())),
        preferred_element_type=jnp.float32)
    counts_ref[0:1, half:] += jax.lax.dot_general(
        take_row, onehot, (((1,), (0,)), ((), ())),
        preferred_element_type=jnp.float32)

    @pl.when(i == nsteps - 1)
    def _finalize():
        mean = lsum_ref[0] / n_total
        loss_ref[...] = jnp.full((1, 1), mean + _BETA * mean, jnp.float32)
        e_mean = counts_ref[...] * (1.0 / (n_total / e.shape[1]))
        ent = -jnp.sum(e_mean * jnp.log(e_mean + 1e-10))
        perp_ref[...] = jnp.full((1, 1), jnp.exp(ent), jnp.float32)


def kernel(z, embedding_weight):
    e_dim = z.shape[-1]
    zf = z.reshape(-1, e_dim)
    m = zf.shape[0]
    n = embedding_weight.shape[0]
    n_total = m * e_dim

    zbf = (-2.0 * zf).astype(jnp.bfloat16)
    zsq = jnp.sum(zf ** 2, axis=1).reshape(m, 1)
    esq = jnp.sum(embedding_weight ** 2, axis=1).reshape(1, n)

    body = functools.partial(_vq_body, n_codes=n, n_total=n_total)
    loss2d, zq_st, idx, perp2d = pl.pallas_call(
        body,
        grid=(m // _MB,),
        in_specs=[
            pl.BlockSpec((_MB, e_dim), lambda i: (i, 0)),
            pl.BlockSpec((_MB, e_dim), lambda i: (i, 0)),
            pl.BlockSpec((_MB, 1), lambda i: (i, 0)),
            pl.BlockSpec((n, e_dim), lambda i: (0, 0)),
            pl.BlockSpec((1, n), lambda i: (0, 0)),
        ],
        out_specs=[
            pl.BlockSpec((1, 1), lambda i: (0, 0)),
            pl.BlockSpec((_MB, e_dim), lambda i: (i, 0)),
            pl.BlockSpec((_MB,), lambda i: (i,)),
            pl.BlockSpec((1, 1), lambda i: (0, 0)),
        ],
        out_shape=[
            jax.ShapeDtypeStruct((1, 1), jnp.float32),
            jax.ShapeDtypeStruct((m, e_dim), jnp.float32),
            jax.ShapeDtypeStruct((m,), jnp.int32),
            jax.ShapeDtypeStruct((1, 1), jnp.float32),
        ],
        scratch_shapes=[
            pltpu.VMEM((1, n), jnp.float32),
            pltpu.SMEM((1,), jnp.float32),
        ],
    )(zf, zbf, zsq, embedding_weight, esq)

    loss = loss2d.reshape(())
    perplexity = perp2d.reshape(())
    return (loss, zq_st.reshape(z.shape), idx, perplexity)


# speedup vs baseline: 1.3656x; 1.0694x over previous
import functools

import jax
import jax.numpy as jnp
from jax.experimental import pallas as pl
from jax.experimental.pallas import tpu as pltpu

_BETA = 0.25
_MB = 1024


def _vq_body(z_ref, zbf_ref, zsq_ref, e_ref, esq_ref,
             loss_ref, zq_ref, idx_ref, perp_ref,
             counts_ref, lsum_ref, *, n_codes, n_total):
    i = pl.program_id(0)
    nsteps = pl.num_programs(0)

    @pl.when(i == 0)
    def _init():
        counts_ref[...] = jnp.zeros_like(counts_ref)
        lsum_ref[0] = 0.0

    zb = z_ref[...]
    zbf = zbf_ref[...]
    e = e_ref[...]
    zsq = zsq_ref[...]
    esq = esq_ref[...]

    mm = jax.lax.dot_general(zbf, e, (((1,), (1,)), ((), ())),
                             preferred_element_type=jnp.float32)
    d = (zsq + esq) + mm

    half = n_codes // 2
    d0 = d[:, :half]
    d1 = d[:, half:]
    m0 = jnp.min(d0, axis=1, keepdims=True)
    m1 = jnp.min(d1, axis=1, keepdims=True)
    m0_bf = m0.astype(jnp.bfloat16).astype(jnp.float32)
    take = m1 < m0_bf

    d_w = jnp.where(take, d1, d0)
    m_w = jnp.where(take, m1, m0)
    jiota = jax.lax.broadcasted_iota(jnp.int32, (zb.shape[0], half), 1)
    i_rel = jnp.min(jnp.where(d_w == m_w, jiota, n_codes), axis=1)
    idx = (i_rel + jnp.where(take[:, 0], half, 0)).astype(jnp.int32)
    idx_ref[...] = idx

    onehot = (jiota == i_rel[:, None]).astype(jnp.float32)
    zq0 = jax.lax.dot_general(onehot, e[:half], (((1,), (0,)), ((), ())),
                              preferred_element_type=jnp.float32)
    zq1 = jax.lax.dot_general(onehot, e[half:], (((1,), (0,)), ((), ())),
                              preferred_element_type=jnp.float32)
    zq = jnp.where(take, zq1, zq0)

    diff = zq - zb
    zq_ref[...] = zb + diff
    lsum_ref[0] += jnp.sum(diff * diff)
    take_row = take.astype(jnp.float32).reshape(1, zb.shape[0])
    keep_row = 1.0 - take_row
    counts_ref[0:1, :half] += jax.lax.dot_general(
        keep_row, onehot, (((1,), (0,)), ((), ())),
        preferred_element_type=jnp.float32)
    counts_ref[0:1, half:] += jax.lax.dot_general(
        take_row, onehot, (((1,), (0,)), ((), ())),
        preferred_element_type=jnp.float32)

    @pl.when(i == nsteps - 1)
    def _finalize():
        mean = lsum_ref[0] / n_total
        loss_ref[...] = jnp.full((1, 1), mean + _BETA * mean, jnp.float32)
        e_mean = counts_ref[...] * (1.0 / (n_total / e.shape[1]))
        ent = -jnp.sum(e_mean * jnp.log(e_mean + 1e-10))
        perp_ref[...] = jnp.full((1, 1), jnp.exp(ent), jnp.float32)


def kernel(z, embedding_weight):
    e_dim = z.shape[-1]
    zf = z.reshape(-1, e_dim)
    m = zf.shape[0]
    n = embedding_weight.shape[0]
    n_total = m * e_dim

    zbf = (-2.0 * zf).astype(jnp.bfloat16)
    zsq = jnp.sum(zf ** 2, axis=1).reshape(m, 1)
    esq = jnp.sum(embedding_weight ** 2, axis=1).reshape(1, n)

    body = functools.partial(_vq_body, n_codes=n, n_total=n_total)
    loss2d, zq_st, idx, perp2d = pl.pallas_call(
        body,
        grid=(m // _MB,),
        in_specs=[
            pl.BlockSpec((_MB, e_dim), lambda i: (i, 0)),
            pl.BlockSpec((_MB, e_dim), lambda i: (i, 0)),
            pl.BlockSpec((_MB, 1), lambda i: (i, 0)),
            pl.BlockSpec((n, e_dim), lambda i: (0, 0)),
            pl.BlockSpec((1, n), lambda i: (0, 0)),
        ],
        out_specs=[
            pl.BlockSpec((1, 1), lambda i: (0, 0)),
            pl.BlockSpec((_MB, e_dim), lambda i: (i, 0)),
            pl.BlockSpec((_MB,), lambda i: (i,)),
            pl.BlockSpec((1, 1), lambda i: (0, 0)),
        ],
        out_shape=[
            jax.ShapeDtypeStruct((1, 1), jnp.float32),
            jax.ShapeDtypeStruct((m, e_dim), jnp.float32),
            jax.ShapeDtypeStruct((m,), jnp.int32),
            jax.ShapeDtypeStruct((1, 1), jnp.float32),
        ],
        scratch_shapes=[
            pltpu.VMEM((1, n), jnp.float32),
            pltpu.SMEM((1,), jnp.float32),
        ],
    )(zf, zbf, zsq, embedding_weight, esq)

    loss = loss2d.reshape(())
    perplexity = perp2d.reshape(())
    return (loss, zq_st.reshape(z.shape), idx, perplexity)


# speedup vs baseline: 1.4497x; 1.0616x over previous
import functools

import jax
import jax.numpy as jnp
from jax import lax
from jax.experimental import pallas as pl
from jax.experimental.pallas import tpu as pltpu
from jax.experimental.pallas import tpu_sc as plsc

_BETA = 0.25
_MB = 1024

_NC, _NS, _LANES = 2, 16, 16
_NW = _NC * _NS
_CH = 128


def _argmin_body(z_ref, zbf_ref, zsq_ref, e_ref, esq_ref,
                 idx_ref, perp_ref, counts_ref, *, n_codes, n_rows):
    i = pl.program_id(0)
    nsteps = pl.num_programs(0)

    @pl.when(i == 0)
    def _init():
        counts_ref[...] = jnp.zeros_like(counts_ref)

    zb = z_ref[...]
    zbf = zbf_ref[...]
    e = e_ref[...]
    zsq = zsq_ref[...]
    esq = esq_ref[...]

    mm = jax.lax.dot_general(zbf, e, (((1,), (1,)), ((), ())),
                             preferred_element_type=jnp.float32)
    d = (zsq + esq) + mm

    half = n_codes // 2
    d0 = d[:, :half]
    d1 = d[:, half:]
    m0 = jnp.min(d0, axis=1, keepdims=True)
    m1 = jnp.min(d1, axis=1, keepdims=True)
    m0_bf = m0.astype(jnp.bfloat16).astype(jnp.float32)
    take = m1 < m0_bf

    d_w = jnp.where(take, d1, d0)
    m_w = jnp.where(take, m1, m0)
    jiota = jax.lax.broadcasted_iota(jnp.int32, (zb.shape[0], half), 1)
    i_rel = jnp.min(jnp.where(d_w == m_w, jiota, n_codes), axis=1)
    idx = (i_rel + jnp.where(take[:, 0], half, 0)).astype(jnp.int32)
    idx_ref[...] = idx

    onehot = (jiota == i_rel[:, None]).astype(jnp.float32)
    take_row = take.astype(jnp.float32).reshape(1, zb.shape[0])
    keep_row = 1.0 - take_row
    counts_ref[0:1, :half] += jax.lax.dot_general(
        keep_row, onehot, (((1,), (0,)), ((), ())),
        preferred_element_type=jnp.float32)
    counts_ref[0:1, half:] += jax.lax.dot_general(
        take_row, onehot, (((1,), (0,)), ((), ())),
        preferred_element_type=jnp.float32)

    @pl.when(i == nsteps - 1)
    def _finalize():
        e_mean = counts_ref[...] * (1.0 / n_rows)
        ent = -jnp.sum(e_mean * jnp.log(e_mean + 1e-10))
        perp_ref[...] = jnp.full((1, 1), jnp.exp(ent), jnp.float32)


def _sc_stage(idx, zf, table_pad):
    m_rows = zf.shape[0]
    bpw = m_rows // _NW
    n_chunks = bpw // _CH

    mesh = plsc.VectorSubcoreMesh(core_axis_name="c", subcore_axis_name="s",
                                  num_cores=_NC)

    @functools.partial(
        pl.kernel, mesh=mesh,
        out_type=jax.ShapeDtypeStruct((m_rows, 32), jnp.float32),
        scratch_types=[
            pltpu.VMEM((_CH,), jnp.int32),
            pltpu.VMEM((_CH, 128), jnp.float32),
            pltpu.VMEM((_CH, 32), jnp.float32),
            pltpu.VMEM((_CH, 32), jnp.float32),
            pltpu.SemaphoreType.DMA,
        ],
    )
    def k(idx_hbm, z_hbm, table_hbm, zq_hbm,
          idx_v, rows_v, z_v, zq_v, sem):
        c = lax.axis_index("c")
        s = lax.axis_index("s")
        wid = s * _NC + c
        base = wid * bpw
        for kc in range(n_chunks):
            off = base + kc * _CH
            pltpu.sync_copy(idx_hbm.at[pl.ds(off, _CH)], idx_v)
            pltpu.async_copy(table_hbm.at[idx_v], rows_v, sem).wait()
            pltpu.sync_copy(z_hbm.at[pl.ds(off, _CH)], z_v)

            def row_body(r, carry):
                z0 = z_v[r, 0:16]
                z1 = z_v[r, 16:32]
                d0 = rows_v[r, 0:16] - z0
                d1 = rows_v[r, 16:32] - z1
                zq_v[r, 0:16] = z0 + d0
                zq_v[r, 16:32] = z1 + d1
                return carry

            lax.fori_loop(0, _CH, row_body, 0)
            pltpu.sync_copy(zq_v, zq_hbm.at[pl.ds(off, _CH)])

    return k(idx, zf, table_pad)


def _loss_body(zq_ref, z_ref, loss_ref, *, n_total):
    diff = zq_ref[...] - z_ref[...]
    mean = jnp.sum(diff * diff) / n_total
    loss_ref[...] = jnp.full((1, 1), mean + _BETA * mean, jnp.float32)


def kernel(z, embedding_weight):
    e_dim = z.shape[-1]
    zf = z.reshape(-1, e_dim)
    m = zf.shape[0]
    n = embedding_weight.shape[0]
    n_total = m * e_dim

    zbf = (-2.0 * zf).astype(jnp.bfloat16)
    zsq = jnp.sum(zf ** 2, axis=1).reshape(m, 1)
    esq = jnp.sum(embedding_weight ** 2, axis=1).reshape(1, n)

    idx, perp2d = pl.pallas_call(
        functools.partial(_argmin_body, n_codes=n, n_rows=m),
        grid=(m // _MB,),
        in_specs=[
            pl.BlockSpec((_MB, e_dim), lambda i: (i, 0)),
            pl.BlockSpec((_MB, e_dim), lambda i: (i, 0)),
            pl.BlockSpec((_MB, 1), lambda i: (i, 0)),
            pl.BlockSpec((n, e_dim), lambda i: (0, 0)),
            pl.BlockSpec((1, n), lambda i: (0, 0)),
        ],
        out_specs=[
            pl.BlockSpec((_MB,), lambda i: (i,)),
            pl.BlockSpec((1, 1), lambda i: (0, 0)),
        ],
        out_shape=[
            jax.ShapeDtypeStruct((m,), jnp.int32),
            jax.ShapeDtypeStruct((1, 1), jnp.float32),
        ],
        scratch_shapes=[
            pltpu.VMEM((1, n), jnp.float32),
        ],
    )(zf, zbf, zsq, embedding_weight, esq)

    table_pad = jnp.pad(embedding_weight, ((0, 0), (0, 128 - e_dim)))
    zq_st = _sc_stage(idx, zf, table_pad)

    loss2d = pl.pallas_call(
        functools.partial(_loss_body, n_total=n_total),
        out_shape=jax.ShapeDtypeStruct((1, 1), jnp.float32),
    )(zq_st, zf)

    loss = loss2d.reshape(())
    perplexity = perp2d.reshape(())
    return (loss, zq_st.reshape(z.shape), idx, perplexity)


# speedup vs baseline: 1.4839x; 1.0236x over previous
import functools

import jax
import jax.numpy as jnp
from jax import lax
from jax.experimental import pallas as pl
from jax.experimental.pallas import tpu as pltpu
from jax.experimental.pallas import tpu_sc as plsc

_BETA = 0.25
_MB = 1024

_NC, _NS, _LANES = 2, 16, 16
_NW = _NC * _NS
_CH = 128


def _argmin_body(z_ref, zbf_ref, zsq_ref, e_ref, esq_ref,
                 idx_ref, perp_ref, counts_ref, *, n_codes, n_rows):
    i = pl.program_id(0)
    nsteps = pl.num_programs(0)

    @pl.when(i == 0)
    def _init():
        counts_ref[...] = jnp.zeros_like(counts_ref)

    zb = z_ref[...]
    zbf = zbf_ref[...]
    e = e_ref[...]
    zsq = zsq_ref[...]
    esq = esq_ref[...]

    mm = jax.lax.dot_general(zbf, e, (((1,), (1,)), ((), ())),
                             preferred_element_type=jnp.float32)
    d = (zsq + esq) + mm

    half = n_codes // 2
    d0 = d[:, :half]
    d1 = d[:, half:]
    m0 = jnp.min(d0, axis=1, keepdims=True)
    m1 = jnp.min(d1, axis=1, keepdims=True)
    m0_bf = m0.astype(jnp.bfloat16).astype(jnp.float32)
    take = m1 < m0_bf

    d_w = jnp.where(take, d1, d0)
    m_w = jnp.where(take, m1, m0)
    jiota = jax.lax.broadcasted_iota(jnp.int32, (zb.shape[0], half), 1)
    i_rel = jnp.min(jnp.where(d_w == m_w, jiota, n_codes), axis=1)
    idx = (i_rel + jnp.where(take[:, 0], half, 0)).astype(jnp.int32)
    idx_ref[...] = idx

    onehot = (jiota == i_rel[:, None]).astype(jnp.float32)
    take_row = take.astype(jnp.float32).reshape(1, zb.shape[0])
    keep_row = 1.0 - take_row
    counts_ref[0:1, :half] += jax.lax.dot_general(
        keep_row, onehot, (((1,), (0,)), ((), ())),
        preferred_element_type=jnp.float32)
    counts_ref[0:1, half:] += jax.lax.dot_general(
        take_row, onehot, (((1,), (0,)), ((), ())),
        preferred_element_type=jnp.float32)

    @pl.when(i == nsteps - 1)
    def _finalize():
        e_mean = counts_ref[...] * (1.0 / n_rows)
        ent = -jnp.sum(e_mean * jnp.log(e_mean + 1e-10))
        perp_ref[...] = jnp.full((1, 1), jnp.exp(ent), jnp.float32)


def _sc_stage(idx, zf, table_pad):
    m_rows = zf.shape[0]
    bpw = m_rows // _NW
    n_chunks = bpw // _CH

    mesh = plsc.VectorSubcoreMesh(core_axis_name="c", subcore_axis_name="s",
                                  num_cores=_NC)

    @functools.partial(
        pl.kernel, mesh=mesh,
        out_type=jax.ShapeDtypeStruct((m_rows, 128), jnp.float32),
        scratch_types=[
            pltpu.VMEM((_CH,), jnp.int32),
            pltpu.VMEM((_CH, 128), jnp.float32),
            pltpu.SemaphoreType.DMA,
        ],
    )
    def k(idx_hbm, z_hbm, table_hbm, zq_hbm,
          idx_v, rows_v, sem):
        c = lax.axis_index("c")
        s = lax.axis_index("s")
        wid = s * _NC + c
        base = wid * bpw
        for kc in range(n_chunks):
            off = base + kc * _CH
            pltpu.sync_copy(idx_hbm.at[pl.ds(off, _CH)], idx_v)
            pltpu.async_copy(table_hbm.at[idx_v], rows_v, sem).wait()
            pltpu.sync_copy(rows_v, zq_hbm.at[pl.ds(off, _CH)])

    return k(idx, zf, table_pad)


def _loss_body(zq_ref, z_ref, loss_ref, *, n_total):
    diff = zq_ref[...] - z_ref[...]
    mean = jnp.sum(diff * diff) / n_total
    loss_ref[...] = jnp.full((1, 1), mean + _BETA * mean, jnp.float32)


def kernel(z, embedding_weight):
    e_dim = z.shape[-1]
    zf = z.reshape(-1, e_dim)
    m = zf.shape[0]
    n = embedding_weight.shape[0]
    n_total = m * e_dim

    zbf = (-2.0 * zf).astype(jnp.bfloat16)
    zsq = jnp.sum(zf ** 2, axis=1).reshape(m, 1)
    esq = jnp.sum(embedding_weight ** 2, axis=1).reshape(1, n)

    idx, perp2d = pl.pallas_call(
        functools.partial(_argmin_body, n_codes=n, n_rows=m),
        grid=(m // _MB,),
        in_specs=[
            pl.BlockSpec((_MB, e_dim), lambda i: (i, 0)),
            pl.BlockSpec((_MB, e_dim), lambda i: (i, 0)),
            pl.BlockSpec((_MB, 1), lambda i: (i, 0)),
            pl.BlockSpec((n, e_dim), lambda i: (0, 0)),
            pl.BlockSpec((1, n), lambda i: (0, 0)),
        ],
        out_specs=[
            pl.BlockSpec((_MB,), lambda i: (i,)),
            pl.BlockSpec((1, 1), lambda i: (0, 0)),
        ],
        out_shape=[
            jax.ShapeDtypeStruct((m,), jnp.int32),
            jax.ShapeDtypeStruct((1, 1), jnp.float32),
        ],
        scratch_shapes=[
            pltpu.VMEM((1, n), jnp.float32),
        ],
    )(zf, zbf, zsq, embedding_weight, esq)

    table_pad = jnp.pad(embedding_weight, ((0, 0), (0, 128 - e_dim)))
    zq_st = _sc_stage(idx, zf, table_pad)[:, :e_dim]

    loss2d = pl.pallas_call(
        functools.partial(_loss_body, n_total=n_total),
        out_shape=jax.ShapeDtypeStruct((1, 1), jnp.float32),
    )(zq_st, zf)

    loss = loss2d.reshape(())
    perplexity = perp2d.reshape(())
    return (loss, zq_st.reshape(z.shape), idx, perplexity)
